# baseline (speedup 1.0000x reference)
import functools

import jax
import jax.numpy as jnp
from jax import lax
from jax.experimental import pallas as pl
from jax.experimental.pallas import tpu as pltpu
from jax.experimental.pallas import tpu_sc as plsc

N_OBJ = 1000
N_REL = 10000
HIDDEN = 512
NUM_OBJ_CLS = 151
NUM_REL_CLS = 51
IOU_THR = 0.3
NEG = -1e30

C_PAD = 152
B_PAD = 1024
NUM_WORKERS = 32
REL_PAD = 10240
ROWS_PER_W = REL_PAD // NUM_WORKERS
CHUNK = 64
NCHUNK = ROWS_PER_W // CHUNK


def _fold_body(wrel_ref, wv_ref, brel_ref, bg_ref, wrv_ref, bias_ref):
    wrv = jnp.dot(wrel_ref[...], wv_ref[...], preferred_element_type=jnp.float32)
    wrv_ref[...] = wrv.astype(jnp.bfloat16)
    bias_ref[...] = (
        jnp.dot(brel_ref[...], wv_ref[...], preferred_element_type=jnp.float32)
        + bg_ref[...]
    )


def _obj_body(fmaps_ref, wobj_ref, bobj_ref, logits_ref, ws_ref, wo_ref,
              wps_ref, wpo_ref, probs_ref, ts_ref, to_ref):
    logits = logits_ref[...]
    m = jnp.max(logits, axis=1, keepdims=True)
    e = jnp.exp(logits - m)
    probs = e / jnp.sum(e, axis=1, keepdims=True)
    probs_ref[...] = probs
    obj_f = jnp.dot(fmaps_ref[...].astype(jnp.bfloat16),
                    wobj_ref[...].astype(jnp.bfloat16),
                    preferred_element_type=jnp.float32) + bobj_ref[...]
    objf_bf = obj_f.astype(jnp.bfloat16)
    probs_bf = probs.astype(jnp.bfloat16)
    ts_ref[...] = (
        jnp.dot(objf_bf, ws_ref[...].astype(jnp.bfloat16),
                preferred_element_type=jnp.float32)
        + jnp.dot(probs_bf, wps_ref[...].astype(jnp.bfloat16),
                  preferred_element_type=jnp.float32))
    to_ref[...] = (
        jnp.dot(objf_bf, wo_ref[...].astype(jnp.bfloat16),
                preferred_element_type=jnp.float32)
        + jnp.dot(probs_bf, wpo_ref[...].astype(jnp.bfloat16),
                  preferred_element_type=jnp.float32))


def _sc_gather(ts, to, s_idx, o_idx):
    mesh = plsc.VectorSubcoreMesh(core_axis_name="c", subcore_axis_name="s")

    @functools.partial(
        pl.kernel,
        mesh=mesh,
        out_type=(
            jax.ShapeDtypeStruct((REL_PAD, HIDDEN), jnp.float32),
            jax.ShapeDtypeStruct((REL_PAD, HIDDEN), jnp.float32),
        ),
        scratch_types=[
            pltpu.VMEM((ROWS_PER_W,), jnp.int32),
            pltpu.VMEM((ROWS_PER_W,), jnp.int32),
            pltpu.VMEM((CHUNK, HIDDEN), jnp.float32),
            pltpu.VMEM((CHUNK, HIDDEN), jnp.float32),
            pltpu.SemaphoreType.DMA,
            pltpu.SemaphoreType.DMA,
        ],
    )
    def body(ts_hbm, to_hbm, sidx_hbm, oidx_hbm, gs_hbm, go_hbm,
             sidx_v, oidx_v, bufs, bufo, sem_s, sem_o):
        wid = lax.axis_index("s") * 2 + lax.axis_index("c")
        base = wid * ROWS_PER_W
        pltpu.sync_copy(sidx_hbm.at[pl.ds(base, ROWS_PER_W)], sidx_v)
        pltpu.sync_copy(oidx_hbm.at[pl.ds(base, ROWS_PER_W)], oidx_v)
        for k in range(NCHUNK):
            cs = pltpu.async_copy(
                ts_hbm.at[sidx_v.at[pl.ds(k * CHUNK, CHUNK)]], bufs, sem_s)
            co = pltpu.async_copy(
                to_hbm.at[oidx_v.at[pl.ds(k * CHUNK, CHUNK)]], bufo, sem_o)
            cs.wait()
            co.wait()
            pltpu.sync_copy(bufs, gs_hbm.at[pl.ds(base + k * CHUNK, CHUNK)])
            pltpu.sync_copy(bufo, go_hbm.at[pl.ds(base + k * CHUNK, CHUNK)])

    return body(ts, to, s_idx, o_idx)


def _rel_body(vr_ref, wrv_ref, gs_ref, go_ref, bias_ref, wcls_ref, bcls_ref,
              out_ref):
    acc = jnp.dot(vr_ref[...].astype(jnp.bfloat16), wrv_ref[...],
                  preferred_element_type=jnp.float32)
    h = jnp.maximum(acc + gs_ref[...] + go_ref[...] + bias_ref[0:1, :], 0.0)
    out_ref[...] = (
        jnp.dot(h, wcls_ref[...], preferred_element_type=jnp.float32)
        + bcls_ref[...])


NCHK = 8
CW = B_PAD // NCHK


def _nms_body(boxes_ref, probs_t_ref, pred_ref, ms_ref, area_ref):
    lane_c = lax.broadcasted_iota(jnp.int32, (C_PAD, CW), 1)
    row_c = lax.broadcasted_iota(jnp.int32, (C_PAD, CW), 0)

    cm0 = []
    for cc in range(NCHK):
        sl = pl.ds(cc * CW, CW)
        pt = probs_t_ref[:, sl]
        v = (row_c >= 1) & (row_c <= 150) & (lane_c + cc * CW < N_OBJ)
        ms = jnp.where(v, pt, NEG)
        ms_ref[:, sl] = ms
        area_ref[:, sl] = ((boxes_ref[2, :, sl] - boxes_ref[0, :, sl])
                           * (boxes_ref[3, :, sl] - boxes_ref[1, :, sl]))
        cm0.append(jnp.max(ms, axis=1, keepdims=True))
    cm0 = jnp.concatenate(cm0, axis=1)

    def cond(c):
        return c[1]

    def body(c):
        cm, _ = c
        m = jnp.max(cm, axis=1, keepdims=True)
        active = m > 0.0

        cand_idx = jnp.full((C_PAD, 1), 4096, jnp.int32)
        cand_x1 = jnp.zeros((C_PAD, 1), jnp.float32)
        cand_y1 = cand_x1
        cand_x2 = cand_x1
        cand_y2 = cand_x1
        for cc in range(NCHK):
            sl = pl.ds(cc * CW, CW)
            ms = ms_ref[:, sl]
            eq = ms == m
            lidx = jnp.min(jnp.where(eq, lane_c, CW), axis=1, keepdims=True)
            sel = (lane_c == lidx)

            def pick(a):
                return jnp.sum(jnp.where(sel, a, 0.0), axis=1, keepdims=True)

            cx1 = pick(boxes_ref[0, :, sl])
            cy1 = pick(boxes_ref[1, :, sl])
            cx2 = pick(boxes_ref[2, :, sl])
            cy2 = pick(boxes_ref[3, :, sl])
            better = (lidx < CW) & (cand_idx >= 4096)
            cand_idx = jnp.where(better, lidx + cc * CW, cand_idx)
            cand_x1 = jnp.where(better, cx1, cand_x1)
            cand_y1 = jnp.where(better, cy1, cand_y1)
            cand_x2 = jnp.where(better, cx2, cand_x2)
            cand_y2 = jnp.where(better, cy2, cand_y2)
        ars = (cand_x2 - cand_x1) * (cand_y2 - cand_y1)

        cms = []
        for cc in range(NCHK):
            sl = pl.ds(cc * CW, CW)
            ms = ms_ref[:, sl]
            bx1 = boxes_ref[0, :, sl]
            by1 = boxes_ref[1, :, sl]
            bx2 = boxes_ref[2, :, sl]
            by2 = boxes_ref[3, :, sl]
            iw = jnp.maximum(jnp.minimum(cand_x2, bx2)
                             - jnp.maximum(cand_x1, bx1), 0.0)
            ih = jnp.maximum(jnp.minimum(cand_y2, by2)
                             - jnp.maximum(cand_y1, by1), 0.0)
            inter = iw * ih
            union = ars + area_ref[:, sl] - inter
            iou = inter / jnp.maximum(union, 1e-10)
            sel = lane_c + cc * CW == cand_idx
            new = jnp.where(sel, -m, jnp.where(iou > IOU_THR, NEG, ms))
            new = jnp.where(active, new, ms)
            ms_ref[:, sl] = new
            cms.append(jnp.max(new, axis=1, keepdims=True))
        return jnp.concatenate(cms, axis=1), jnp.any(active)

    lax.while_loop(cond, body, (cm0, True))

    for cc in range(NCHK):
        sl = pl.ds(cc * CW, CW)
        masked = jnp.where(ms_ref[:, sl] > -1e29, probs_t_ref[:, sl], 0.0)
        v = (row_c >= 1) & (row_c <= 150) & (lane_c + cc * CW < N_OBJ)
        mv = jnp.where(v, masked, -1.0)
        m2 = jnp.max(mv, axis=0, keepdims=True)
        cls = jnp.min(jnp.where(mv == m2, row_c, 2048), axis=0, keepdims=True)
        pred_ref[:, sl] = jnp.broadcast_to(cls, (8, CW))


def kernel(im_inds, obj_fmaps, obj_logits, rel_inds, vr, boxes_per_cls,
           W_obj_proj, b_obj_proj, W_rel_proj, b_rel_proj,
           W_s, W_o, W_v, W_ps, W_po, b_g, W_cls, b_cls):
    f32 = jnp.float32

    brel2d = jnp.broadcast_to(b_rel_proj[None, :], (8, HIDDEN))
    bg2d = jnp.broadcast_to(b_g[None, :], (8, HIDDEN))
    wrv_bf, bias_h = pl.pallas_call(
        _fold_body,
        out_shape=(
            jax.ShapeDtypeStruct((4096, HIDDEN), jnp.bfloat16),
            jax.ShapeDtypeStruct((8, HIDDEN), f32),
        ),
    )(W_rel_proj, W_v, brel2d, bg2d)

    probs, ts, to = pl.pallas_call(
        _obj_body,
        out_shape=(
            jax.ShapeDtypeStruct((N_OBJ, NUM_OBJ_CLS), f32),
            jax.ShapeDtypeStruct((N_OBJ, HIDDEN), f32),
            jax.ShapeDtypeStruct((N_OBJ, HIDDEN), f32),
        ),
    )(obj_fmaps, W_obj_proj, b_obj_proj[None, :], obj_logits,
      W_s, W_o, W_ps, W_po)

    pad = jnp.zeros((REL_PAD - N_REL,), jnp.int32)
    s_idx = jnp.concatenate([rel_inds[:, 1], pad])
    o_idx = jnp.concatenate([rel_inds[:, 2], pad])
    gs, go = _sc_gather(ts, to, s_idx, o_idx)

    rblk = 400
    rel_logits = pl.pallas_call(
        _rel_body,
        grid=(N_REL // rblk,),
        in_specs=[
            pl.BlockSpec((rblk, 4096), lambda i: (i, 0)),
            pl.BlockSpec((4096, HIDDEN), lambda i: (0, 0)),
            pl.BlockSpec((rblk, HIDDEN), lambda i: (i, 0)),
            pl.BlockSpec((rblk, HIDDEN), lambda i: (i, 0)),
            pl.BlockSpec((8, HIDDEN), lambda i: (0, 0)),
            pl.BlockSpec((HIDDEN, NUM_REL_CLS), lambda i: (0, 0)),
            pl.BlockSpec((1, NUM_REL_CLS), lambda i: (0, 0)),
        ],
        out_specs=pl.BlockSpec((rblk, NUM_REL_CLS), lambda i: (i, 0)),
        out_shape=jax.ShapeDtypeStruct((N_REL, NUM_REL_CLS), f32),
    )(vr, wrv_bf, gs, go, bias_h, W_cls, b_cls[None, :])

    probs_t = jnp.pad(probs.T, ((0, C_PAD - NUM_OBJ_CLS), (0, B_PAD - N_OBJ)))
    boxes_t = jnp.pad(jnp.transpose(boxes_per_cls, (2, 1, 0)),
                      ((0, 0), (0, C_PAD - NUM_OBJ_CLS), (0, B_PAD - N_OBJ)))
    pred8 = pl.pallas_call(
        _nms_body,
        out_shape=jax.ShapeDtypeStruct((8, B_PAD), jnp.int32),
        scratch_shapes=[
            pltpu.VMEM((C_PAD, B_PAD), jnp.float32),
            pltpu.VMEM((C_PAD, B_PAD), jnp.float32),
        ],
    )(boxes_t, probs_t)
    obj_preds = pred8[0, :N_OBJ]

    return obj_logits, obj_preds, rel_logits


# speedup vs baseline: 2.2732x; 2.2732x over previous
import functools

import jax
import jax.numpy as jnp
from jax import lax
from jax.experimental import pallas as pl
from jax.experimental.pallas import tpu as pltpu
from jax.experimental.pallas import tpu_sc as plsc

N_OBJ = 1000
N_REL = 10000
HIDDEN = 512
NUM_OBJ_CLS = 151
NUM_REL_CLS = 51
IOU_THR = 0.3
NEG = -1e30

C_PAD = 152
B_PAD = 1024
NUM_WORKERS = 32
REL_PAD = 10240
ROWS_PER_W = REL_PAD // NUM_WORKERS
CHUNK = 40
NCHUNK = ROWS_PER_W // CHUNK


def _fold_body(wrel_ref, wv_ref, brel_ref, bg_ref, wrv_ref, bias_ref):
    wrv = jnp.dot(wrel_ref[...].astype(jnp.bfloat16),
                  wv_ref[...].astype(jnp.bfloat16),
                  preferred_element_type=jnp.float32)
    wrv_ref[...] = wrv.astype(jnp.bfloat16)
    bias_ref[...] = (
        jnp.dot(brel_ref[...], wv_ref[...], preferred_element_type=jnp.float32)
        + bg_ref[...]
    )


def _obj_body(fmaps_ref, wobj_ref, bobj_ref, logits_ref, ws_ref, wo_ref,
              wps_ref, wpo_ref, probs_ref, ts_ref, to_ref):
    logits = logits_ref[...]
    m = jnp.max(logits, axis=1, keepdims=True)
    e = jnp.exp(logits - m)
    probs = e / jnp.sum(e, axis=1, keepdims=True)
    probs_ref[...] = probs
    obj_f = jnp.dot(fmaps_ref[...].astype(jnp.bfloat16),
                    wobj_ref[...].astype(jnp.bfloat16),
                    preferred_element_type=jnp.float32) + bobj_ref[...]
    objf_bf = obj_f.astype(jnp.bfloat16)
    probs_bf = probs.astype(jnp.bfloat16)
    ts_ref[...] = (
        jnp.dot(objf_bf, ws_ref[...].astype(jnp.bfloat16),
                preferred_element_type=jnp.float32)
        + jnp.dot(probs_bf, wps_ref[...].astype(jnp.bfloat16),
                  preferred_element_type=jnp.float32))
    to_ref[...] = (
        jnp.dot(objf_bf, wo_ref[...].astype(jnp.bfloat16),
                preferred_element_type=jnp.float32)
        + jnp.dot(probs_bf, wpo_ref[...].astype(jnp.bfloat16),
                  preferred_element_type=jnp.float32))


def _sc_gather(ts, to, s_idx, o_idx):
    mesh = plsc.VectorSubcoreMesh(core_axis_name="c", subcore_axis_name="s")

    @functools.partial(
        pl.kernel,
        mesh=mesh,
        out_type=(
            jax.ShapeDtypeStruct((REL_PAD, HIDDEN), jnp.float32),
            jax.ShapeDtypeStruct((REL_PAD, HIDDEN), jnp.float32),
        ),
        scratch_types=[
            pltpu.VMEM((ROWS_PER_W,), jnp.int32),
            pltpu.VMEM((ROWS_PER_W,), jnp.int32),
            pltpu.VMEM((CHUNK, HIDDEN), jnp.float32),
            pltpu.VMEM((CHUNK, HIDDEN), jnp.float32),
            pltpu.VMEM((CHUNK, HIDDEN), jnp.float32),
            pltpu.VMEM((CHUNK, HIDDEN), jnp.float32),
            pltpu.SemaphoreType.DMA,
            pltpu.SemaphoreType.DMA,
            pltpu.SemaphoreType.DMA,
            pltpu.SemaphoreType.DMA,
            pltpu.SemaphoreType.DMA,
            pltpu.SemaphoreType.DMA,
            pltpu.SemaphoreType.DMA,
            pltpu.SemaphoreType.DMA,
        ],
    )
    def body(ts_hbm, to_hbm, sidx_hbm, oidx_hbm, gs_hbm, go_hbm,
             sidx_v, oidx_v, bufs0, bufs1, bufo0, bufo1,
             gs0, gs1, go0, go1, ws0, ws1, wo0, wo1):
        wid = lax.axis_index("s") * 2 + lax.axis_index("c")
        base = wid * ROWS_PER_W
        pltpu.sync_copy(sidx_hbm.at[pl.ds(base, ROWS_PER_W)], sidx_v)
        pltpu.sync_copy(oidx_hbm.at[pl.ds(base, ROWS_PER_W)], oidx_v)
        bufs = (bufs0, bufs1)
        bufo = (bufo0, bufo1)
        gsem = (gs0, gs1)
        osem = (go0, go1)
        wssem = (ws0, ws1)
        wosem = (wo0, wo1)

        def gstart(k):
            b = k % 2
            isl = pl.ds(k * CHUNK, CHUNK)
            return (
                pltpu.async_copy(ts_hbm.at[sidx_v.at[isl]], bufs[b], gsem[b]),
                pltpu.async_copy(to_hbm.at[oidx_v.at[isl]], bufo[b], osem[b]),
            )

        gd = [None] * NCHUNK
        wd = [None] * NCHUNK
        gd[0] = gstart(0)
        for k in range(NCHUNK):
            b = k % 2
            if k + 1 < NCHUNK:
                if k >= 1:
                    for w in wd[k - 1]:
                        w.wait()
                gd[k + 1] = gstart(k + 1)
            for g in gd[k]:
                g.wait()
            rows = pl.ds(base + k * CHUNK, CHUNK)
            wd[k] = (
                pltpu.async_copy(bufs[b], gs_hbm.at[rows], wssem[b]),
                pltpu.async_copy(bufo[b], go_hbm.at[rows], wosem[b]),
            )
        for k in (NCHUNK - 2, NCHUNK - 1):
            for w in wd[k]:
                w.wait()

    return body(ts, to, s_idx, o_idx)


def _rel_body(vr_ref, wrv_ref, gs_ref, go_ref, bias_ref, wcls_ref, bcls_ref,
              out_ref):
    acc = jnp.dot(vr_ref[...].astype(jnp.bfloat16), wrv_ref[...],
                  preferred_element_type=jnp.float32)
    h = jnp.maximum(acc + gs_ref[...] + go_ref[...] + bias_ref[0:1, :], 0.0)
    out_ref[...] = (
        jnp.dot(h, wcls_ref[...], preferred_element_type=jnp.float32)
        + bcls_ref[...])


def _nms_body(boxes_ref, probs_t_ref, pred_ref):
    x1 = boxes_ref[0]
    y1 = boxes_ref[1]
    x2 = boxes_ref[2]
    y2 = boxes_ref[3]
    lane = lax.broadcasted_iota(jnp.int32, (C_PAD, B_PAD), 1)
    row = lax.broadcasted_iota(jnp.int32, (C_PAD, B_PAD), 0)
    valid = (row >= 1) & (row <= 150) & (lane < N_OBJ)
    probs_t = probs_t_ref[...]
    mscore0 = jnp.where(valid, probs_t, NEG)
    areas = (x2 - x1) * (y2 - y1)

    def cond(c):
        return c[1]

    def body(c):
        mscore, _ = c
        m = jnp.max(mscore, axis=1, keepdims=True)
        active = m > 0.0
        eq = mscore == m
        sel_idx = jnp.min(jnp.where(eq, lane, 2048), axis=1, keepdims=True)
        sel = lane == sel_idx

        def pick(a):
            return jnp.sum(jnp.where(sel, a, 0.0), axis=1, keepdims=True)

        x1s = pick(x1)
        y1s = pick(y1)
        x2s = pick(x2)
        y2s = pick(y2)
        ars = (x2s - x1s) * (y2s - y1s)
        iw = jnp.maximum(jnp.minimum(x2s, x2) - jnp.maximum(x1s, x1), 0.0)
        ih = jnp.maximum(jnp.minimum(y2s, y2) - jnp.maximum(y1s, y1), 0.0)
        inter = iw * ih
        union = ars + areas - inter
        iou = inter / jnp.maximum(union, 1e-10)
        new = jnp.where(sel, -m, jnp.where(iou > IOU_THR, NEG, mscore))
        mscore = jnp.where(active, new, mscore)
        return mscore, jnp.any(active)

    mscore, _ = lax.while_loop(cond, body, (mscore0, True))

    masked = jnp.where(mscore > -1e29, probs_t, 0.0)
    mv = jnp.where(valid, masked, -1.0)
    m2 = jnp.max(mv, axis=0, keepdims=True)
    cls = jnp.min(jnp.where(mv == m2, row, 2048), axis=0, keepdims=True)
    pred_ref[...] = jnp.broadcast_to(cls, (8, B_PAD))


def kernel(im_inds, obj_fmaps, obj_logits, rel_inds, vr, boxes_per_cls,
           W_obj_proj, b_obj_proj, W_rel_proj, b_rel_proj,
           W_s, W_o, W_v, W_ps, W_po, b_g, W_cls, b_cls):
    f32 = jnp.float32

    brel2d = jnp.broadcast_to(b_rel_proj[None, :], (8, HIDDEN))
    bg2d = jnp.broadcast_to(b_g[None, :], (8, HIDDEN))
    wrv_bf, bias_h = pl.pallas_call(
        _fold_body,
        out_shape=(
            jax.ShapeDtypeStruct((4096, HIDDEN), jnp.bfloat16),
            jax.ShapeDtypeStruct((8, HIDDEN), f32),
        ),
    )(W_rel_proj, W_v, brel2d, bg2d)

    probs, ts, to = pl.pallas_call(
        _obj_body,
        out_shape=(
            jax.ShapeDtypeStruct((N_OBJ, NUM_OBJ_CLS), f32),
            jax.ShapeDtypeStruct((N_OBJ, HIDDEN), f32),
            jax.ShapeDtypeStruct((N_OBJ, HIDDEN), f32),
        ),
    )(obj_fmaps, W_obj_proj, b_obj_proj[None, :], obj_logits,
      W_s, W_o, W_ps, W_po)

    pad = jnp.zeros((REL_PAD - N_REL,), jnp.int32)
    s_idx = jnp.concatenate([rel_inds[:, 1], pad])
    o_idx = jnp.concatenate([rel_inds[:, 2], pad])
    gs, go = _sc_gather(ts, to, s_idx, o_idx)

    rblk = 400
    rel_logits = pl.pallas_call(
        _rel_body,
        grid=(N_REL // rblk,),
        in_specs=[
            pl.BlockSpec((rblk, 4096), lambda i: (i, 0)),
            pl.BlockSpec((4096, HIDDEN), lambda i: (0, 0)),
            pl.BlockSpec((rblk, HIDDEN), lambda i: (i, 0)),
            pl.BlockSpec((rblk, HIDDEN), lambda i: (i, 0)),
            pl.BlockSpec((8, HIDDEN), lambda i: (0, 0)),
            pl.BlockSpec((HIDDEN, NUM_REL_CLS), lambda i: (0, 0)),
            pl.BlockSpec((1, NUM_REL_CLS), lambda i: (0, 0)),
        ],
        out_specs=pl.BlockSpec((rblk, NUM_REL_CLS), lambda i: (i, 0)),
        out_shape=jax.ShapeDtypeStruct((N_REL, NUM_REL_CLS), f32),
    )(vr, wrv_bf, gs, go, bias_h, W_cls, b_cls[None, :])

    probs_t = jnp.pad(probs.T, ((0, C_PAD - NUM_OBJ_CLS), (0, B_PAD - N_OBJ)))
    boxes_t = jnp.pad(jnp.transpose(boxes_per_cls, (2, 1, 0)),
                      ((0, 0), (0, C_PAD - NUM_OBJ_CLS), (0, B_PAD - N_OBJ)))
    pred8 = pl.pallas_call(
        _nms_body,
        out_shape=jax.ShapeDtypeStruct((8, B_PAD), jnp.int32),
    )(boxes_t, probs_t)
    obj_preds = pred8[0, :N_OBJ]

    return obj_logits, obj_preds, rel_logits
